# baseline (speedup 1.0000x reference)
import functools

import jax
import jax.numpy as jnp
from jax import lax
from jax.experimental import pallas as pl
from jax.experimental.pallas import tpu as pltpu
from jax.experimental.pallas import tpu_sc as plsc

HIDDEN_DIM = 64
BETA = 5.0
LANES = 16
NUM_WORKERS = 32
BSLAB = 128


def _sc_boost(emb_t, ids_t, table, weights):
    num_l, d, batch = emb_t.shape
    assert d == HIDDEN_DIM and batch == NUM_WORKERS * BSLAB and num_l % 2 == 0
    mesh = plsc.VectorSubcoreMesh(core_axis_name="c", subcore_axis_name="s")

    @functools.partial(
        pl.kernel,
        out_type=jax.ShapeDtypeStruct(emb_t.shape, jnp.float32),
        mesh=mesh,
        compiler_params=pltpu.CompilerParams(needs_layout_passes=False),
        scratch_types=[
            pltpu.VMEM((num_l, BSLAB), jnp.int32),
            pltpu.VMEM((BSLAB,), jnp.int32),
            pltpu.VMEM((BSLAB,), jnp.int32),
            pltpu.VMEM((BSLAB,), jnp.float32),
            pltpu.VMEM((BSLAB,), jnp.float32),
            pltpu.VMEM((BSLAB, 2 * HIDDEN_DIM), jnp.float32),
            pltpu.VMEM((BSLAB, 2 * HIDDEN_DIM), jnp.float32),
            pltpu.VMEM((HIDDEN_DIM, BSLAB), jnp.float32),
            pltpu.VMEM((HIDDEN_DIM, BSLAB), jnp.float32),
            pltpu.SemaphoreType.DMA,
            pltpu.SemaphoreType.DMA,
            pltpu.SemaphoreType.DMA,
            pltpu.SemaphoreType.DMA,
        ],
    )
    def k(emb_hbm, ids_hbm, tab_hbm, w_hbm, out_hbm,
          ids_v, ri0, ri1, w0, w1, rows0, rows1, eb0, eb1,
          sem_in0, sem_in1, sem_out0, sem_out1):
        num_cores = jax.lax.axis_size("c")
        wid = lax.axis_index("s") * num_cores + lax.axis_index("c")
        b0 = wid * BSLAB
        pltpu.sync_copy(ids_hbm.at[:, pl.ds(b0, BSLAB)], ids_v)

        bufs = ((ri0, w0, rows0, eb0, sem_in0, sem_out0),
                (ri1, w1, rows1, eb1, sem_in1, sem_out1))

        def prep_idx(b, step):
            ri_v = bufs[b][0]
            for jg in range(BSLAB // LANES):
                sl = pl.ds(jg * LANES, LANES)
                ri_v[sl] = lax.shift_right_logical(ids_v[step, sl], 1)

        def issue_in(b, step):
            ri_v, w_v, rows_v, eb_v, sem_in, _ = bufs[b]
            pltpu.async_copy(tab_hbm.at[ri_v], rows_v, sem_in)
            pltpu.async_copy(w_hbm.at[ids_v.at[step]], w_v, sem_in)
            pltpu.async_copy(
                emb_hbm.at[step, :, pl.ds(b0, BSLAB)], eb_v, sem_in)

        def wait_in(b, step):
            ri_v, w_v, rows_v, eb_v, sem_in, _ = bufs[b]
            pltpu.make_async_copy(tab_hbm.at[ri_v], rows_v, sem_in).wait()
            pltpu.make_async_copy(w_hbm.at[ids_v.at[step]], w_v, sem_in).wait()
            pltpu.make_async_copy(
                emb_hbm.at[step, :, pl.ds(b0, BSLAB)], eb_v, sem_in).wait()

        def issue_out(b, step):
            eb_v, sem_out = bufs[b][3], bufs[b][5]
            pltpu.async_copy(eb_v, out_hbm.at[step, :, pl.ds(b0, BSLAB)],
                             sem_out)

        def wait_out(b, step):
            eb_v, sem_out = bufs[b][3], bufs[b][5]
            pltpu.make_async_copy(
                eb_v, out_hbm.at[step, :, pl.ds(b0, BSLAB)], sem_out).wait()

        def compute(b, step):
            ri_v, w_v, rows_v, eb_v, _, _ = bufs[b]
            iota = lax.iota(jnp.int32, LANES)
            for jg in range(BSLAB // LANES):
                sl = pl.ds(jg * LANES, LANES)
                idv = ids_v[step, sl]
                pofs = (idv & 1) * HIDDEN_DIM
                rowi = iota + (jg * LANES)
                sj = w_v[sl] * BETA

                def d4(d4i, colv):
                    for u in range(4):
                        dd = d4i * 4 + u
                        val = plsc.load_gather(rows_v, [rowi, colv])
                        eb_v[dd, sl] = eb_v[dd, sl] + sj * val
                        colv = colv + 1
                    return colv

                lax.fori_loop(0, HIDDEN_DIM // 4, d4, pofs)

        prep_idx(0, 0)
        issue_in(0, 0)

        def pair(ii, _):
            for b in range(2):
                step = 2 * ii + b
                o = 1 - b

                @pl.when(step + 1 < num_l)
                def _():
                    @pl.when(step >= 1)
                    def _():
                        wait_out(o, step)
                    prep_idx(o, step + 1)
                    issue_in(o, step + 1)

                wait_in(b, step)
                compute(b, step)
                issue_out(b, step)
            return 0

        lax.fori_loop(0, num_l // 2, pair, 0)
        wait_out(0, num_l - 2)
        wait_out(1, num_l - 1)

    return k(emb_t, ids_t, table, weights)


def kernel(embeddings, token_ids, agency_matrix, boosting_weights):
    b, l, d = embeddings.shape
    emb_t = jnp.transpose(embeddings, (1, 2, 0))
    ids_t = jnp.transpose(token_ids, (1, 0)).astype(jnp.int32)
    table = agency_matrix.reshape(-1, 2 * d)
    out_t = _sc_boost(emb_t, ids_t, table, boosting_weights)
    return jnp.transpose(out_t, (2, 0, 1))
